# baseline (speedup 1.0000x reference)
import jax
import jax.numpy as jnp
from jax import lax
from jax.experimental import pallas as pl
from jax.experimental.pallas import tpu as pltpu
from jax.experimental.pallas import tpu_sc as plsc

_LUT_SIZE = 1024
_MAX_DISTANCE = 10.0
_SCALE = (_LUT_SIZE - 1) / _MAX_DISTANCE

_NUM_CORES = 2
_NUM_SUBCORES = 16
_NW = _NUM_CORES * _NUM_SUBCORES
_LANES = 16
_CHUNK_COLS = 128


def _build_diff(lut_hbm, base_vmem, diff_vmem):
    pltpu.sync_copy(lut_hbm, base_vmem.at[pl.ds(0, _LUT_SIZE)])
    last = plsc.load_gather(
        base_vmem, [jnp.full((_LANES,), _LUT_SIZE - 1, jnp.int32)])
    base_vmem[pl.ds(_LUT_SIZE, _LANES)] = last

    @plsc.parallel_loop(0, _LUT_SIZE, step=_LANES)
    def _(i):
        a = base_vmem[pl.ds(i, _LANES)]
        b = base_vmem[pl.ds(i + 1, _LANES)]
        diff_vmem[pl.ds(i, _LANES)] = b - a


def _interp_chunk(d_vmem, base_vmem, diff_vmem, o_vmem, rows):
    @plsc.parallel_loop(0, rows, step=1, unroll=2)
    def _(r):
        for j in range(_CHUNK_COLS // _LANES):
            d = d_vmem[r, pl.ds(j * _LANES, _LANES)]
            x = d * _SCALE
            i0 = x.astype(jnp.int32)
            frac = x - i0.astype(jnp.float32)
            v0 = plsc.load_gather(base_vmem, [i0])
            s = plsc.load_gather(diff_vmem, [i0])
            o_vmem[r, pl.ds(j * _LANES, _LANES)] = v0 + frac * s


def _make_sc_kernel(rows, cols):
    cols_per_w = cols // _NW
    n_chunks = cols_per_w // _CHUNK_COLS
    mesh = plsc.VectorSubcoreMesh(
        core_axis_name="c", subcore_axis_name="s",
        num_cores=_NUM_CORES, num_subcores=_NUM_SUBCORES)

    n_pairs = n_chunks // 2

    def body(d_hbm, lut_hbm, out_hbm, base_vmem, diff_vmem,
             d_v0, d_v1, o_v0, o_v1, sin0, sin1, sout0, sout1):
        wid = lax.axis_index("s") * _NUM_CORES + lax.axis_index("c")
        c_base = wid * cols_per_w
        d_bufs, o_bufs = (d_v0, d_v1), (o_v0, o_v1)
        sin, sout = (sin0, sin1), (sout0, sout1)

        def in_desc(c, b):
            return pltpu.make_async_copy(
                d_hbm.at[:, pl.ds(c_base + c * _CHUNK_COLS, _CHUNK_COLS)],
                d_bufs[b], sin[b])

        def out_desc(c, b):
            return pltpu.make_async_copy(
                o_bufs[b],
                out_hbm.at[:, pl.ds(c_base + c * _CHUNK_COLS, _CHUNK_COLS)],
                sout[b])

        def in_copy(c, b):
            in_desc(c, b).start()

        def out_copy(c, b):
            out_desc(c, b).start()

        in_copy(0, 0)
        in_copy(1, 1)
        _build_diff(lut_hbm, base_vmem, diff_vmem)

        def pair(p, _):
            for b in range(2):
                c = 2 * p + b
                in_desc(c, b).wait()

                @pl.when(p > 0)
                def _():
                    out_desc(c - 2, b).wait()

                _interp_chunk(d_bufs[b], base_vmem, diff_vmem,
                              o_bufs[b], rows)
                out_copy(c, b)

                @pl.when(p < n_pairs - 1)
                def _():
                    in_copy(c + 2, b)
            return ()

        lax.fori_loop(0, n_pairs, pair, ())
        last = 2 * (n_pairs - 1)
        out_desc(last, 0).wait()
        out_desc(last + 1, 1).wait()

    return pl.kernel(
        body,
        out_type=jax.ShapeDtypeStruct((rows, cols), jnp.float32),
        mesh=mesh,
        compiler_params=pltpu.CompilerParams(needs_layout_passes=False),
        scratch_types=[
            pltpu.VMEM((_LUT_SIZE + _LANES,), jnp.float32),
            pltpu.VMEM((_LUT_SIZE,), jnp.float32),
            pltpu.VMEM((rows, _CHUNK_COLS), jnp.float32),
            pltpu.VMEM((rows, _CHUNK_COLS), jnp.float32),
            pltpu.VMEM((rows, _CHUNK_COLS), jnp.float32),
            pltpu.VMEM((rows, _CHUNK_COLS), jnp.float32),
            pltpu.SemaphoreType.DMA,
            pltpu.SemaphoreType.DMA,
            pltpu.SemaphoreType.DMA,
            pltpu.SemaphoreType.DMA,
        ],
    )


def kernel(distances, lut):
    rows, cols = distances.shape
    return _make_sc_kernel(cols, rows)(distances.T, lut).T


# speedup vs baseline: 1.0663x; 1.0663x over previous
import jax
import jax.numpy as jnp
from jax import lax
from jax.experimental import pallas as pl
from jax.experimental.pallas import tpu as pltpu
from jax.experimental.pallas import tpu_sc as plsc

_LUT_SIZE = 1024
_MAX_DISTANCE = 10.0
_SCALE = (_LUT_SIZE - 1) / _MAX_DISTANCE

_NUM_CORES = 2
_NUM_SUBCORES = 16
_NW = _NUM_CORES * _NUM_SUBCORES
_LANES = 16
_CHUNK_COLS = 128


def _build_diff(lut_hbm, base_vmem, diff_vmem):
    pltpu.sync_copy(lut_hbm, base_vmem.at[pl.ds(0, _LUT_SIZE)])
    last = plsc.load_gather(
        base_vmem, [jnp.full((_LANES,), _LUT_SIZE - 1, jnp.int32)])
    base_vmem[pl.ds(_LUT_SIZE, _LANES)] = last

    @plsc.parallel_loop(0, _LUT_SIZE, step=_LANES)
    def _(i):
        a = base_vmem[pl.ds(i, _LANES)]
        b = base_vmem[pl.ds(i + 1, _LANES)]
        diff_vmem[pl.ds(i, _LANES)] = b - a


def _interp_chunk(d_vmem, base_vmem, diff_vmem, o_vmem, rows):
    @plsc.parallel_loop(0, rows, step=1, unroll=1)
    def _(r):
        for j in range(_CHUNK_COLS // _LANES):
            d = d_vmem[r, pl.ds(j * _LANES, _LANES)]
            x = d * _SCALE
            i0 = x.astype(jnp.int32)
            frac = x - i0.astype(jnp.float32)
            v0 = plsc.load_gather(base_vmem, [i0])
            s = plsc.load_gather(diff_vmem, [i0])
            o_vmem[r, pl.ds(j * _LANES, _LANES)] = v0 + frac * s


def _make_sc_kernel(rows, cols):
    cols_per_w = cols // _NW
    n_chunks = cols_per_w // _CHUNK_COLS
    mesh = plsc.VectorSubcoreMesh(
        core_axis_name="c", subcore_axis_name="s",
        num_cores=_NUM_CORES, num_subcores=_NUM_SUBCORES)

    n_pairs = n_chunks // 2

    def body(d_hbm, lut_hbm, out_hbm, base_vmem, diff_vmem,
             d_v0, d_v1, o_v0, o_v1, sin0, sin1, sout0, sout1):
        wid = lax.axis_index("s") * _NUM_CORES + lax.axis_index("c")
        c_base = wid * cols_per_w
        d_bufs, o_bufs = (d_v0, d_v1), (o_v0, o_v1)
        sin, sout = (sin0, sin1), (sout0, sout1)

        def in_desc(c, b):
            return pltpu.make_async_copy(
                d_hbm.at[:, pl.ds(c_base + c * _CHUNK_COLS, _CHUNK_COLS)],
                d_bufs[b], sin[b])

        def out_desc(c, b):
            return pltpu.make_async_copy(
                o_bufs[b],
                out_hbm.at[:, pl.ds(c_base + c * _CHUNK_COLS, _CHUNK_COLS)],
                sout[b])

        def in_copy(c, b):
            in_desc(c, b).start()

        def out_copy(c, b):
            out_desc(c, b).start()

        in_copy(0, 0)
        in_copy(1, 1)
        _build_diff(lut_hbm, base_vmem, diff_vmem)

        def pair(p, _):
            for b in range(2):
                c = 2 * p + b
                in_desc(c, b).wait()

                @pl.when(p > 0)
                def _():
                    out_desc(c - 2, b).wait()

                _interp_chunk(d_bufs[b], base_vmem, diff_vmem,
                              o_bufs[b], rows)
                out_copy(c, b)

                @pl.when(p < n_pairs - 1)
                def _():
                    in_copy(c + 2, b)
            return ()

        lax.fori_loop(0, n_pairs, pair, ())
        last = 2 * (n_pairs - 1)
        out_desc(last, 0).wait()
        out_desc(last + 1, 1).wait()

    return pl.kernel(
        body,
        out_type=jax.ShapeDtypeStruct((rows, cols), jnp.float32),
        mesh=mesh,
        compiler_params=pltpu.CompilerParams(needs_layout_passes=False),
        scratch_types=[
            pltpu.VMEM((_LUT_SIZE + _LANES,), jnp.float32),
            pltpu.VMEM((_LUT_SIZE,), jnp.float32),
            pltpu.VMEM((rows, _CHUNK_COLS), jnp.float32),
            pltpu.VMEM((rows, _CHUNK_COLS), jnp.float32),
            pltpu.VMEM((rows, _CHUNK_COLS), jnp.float32),
            pltpu.VMEM((rows, _CHUNK_COLS), jnp.float32),
            pltpu.SemaphoreType.DMA,
            pltpu.SemaphoreType.DMA,
            pltpu.SemaphoreType.DMA,
            pltpu.SemaphoreType.DMA,
        ],
    )


def kernel(distances, lut):
    rows, cols = distances.shape
    return _make_sc_kernel(cols, rows)(distances.T, lut).T
